# Initial kernel scaffold; baseline (speedup 1.0000x reference)
import jax
import jax.numpy as jnp
from jax.experimental import pallas as pl


def kernel(x, edge_index, edge_attr, W_e1, b_e1, W_e2, b_e2, g_e, bb_e, eps_e, W_f1, W_f2, eps_n, W_n1, b_n1, W_n2, b_n2, g_n, bb_n, W_p1, b_p1, W_p2, b_p2):
    raise NotImplementedError("write your pallas kernel here")


# speedup vs baseline: 1.9473x; 1.9473x over previous
import functools

import jax
import jax.numpy as jnp
from jax import lax
from jax.experimental import pallas as pl
from jax.experimental.pallas import tpu as pltpu
from jax.experimental.pallas import tpu_sc as plsc

N_NODES = 10000
N_EDGES = 80000
D_NODE = 32
D_EDGE = 16
D_HID = 32
D_PAD = 128

NC = 2
NS = 16
NW = NC * NS
CHUNK = 128
N_CHUNKS = N_EDGES // CHUNK
MAX_CHUNKS_PER_W = -(-N_CHUNKS // NW)
ROWS_PER_SUB = 624
TAIL_ROWS = N_NODES - NS * ROWS_PER_SUB
ZBUF_ROWS = 48

_sc_mesh = functools.partial(
    plsc.VectorSubcoreMesh, core_axis_name="c", subcore_axis_name="s",
    num_cores=NC, num_subcores=NS)


def _gather_body(x_hbm, src_hbm, out_hbm, idx_v, rows_v, sem):
    wid = lax.axis_index("s") * NC + lax.axis_index("c")

    @pl.loop(0, MAX_CHUNKS_PER_W)
    def _(i):
        chunk = wid + i * NW

        @pl.when(chunk < N_CHUNKS)
        def _():
            base = chunk * CHUNK
            pltpu.sync_copy(src_hbm.at[pl.ds(base, CHUNK)], idx_v)
            pltpu.async_copy(x_hbm.at[idx_v], rows_v, sem).wait()
            pltpu.sync_copy(rows_v, out_hbm.at[pl.ds(base, CHUNK)])


def _sc_gather(x_pad, src):
    k = pl.kernel(
        _gather_body,
        out_type=jax.ShapeDtypeStruct((N_EDGES, D_PAD), jnp.float32),
        mesh=_sc_mesh(),
        scratch_types=[
            pltpu.VMEM((CHUNK,), jnp.int32),
            pltpu.VMEM((CHUNK, D_PAD), jnp.float32),
            pltpu.SemaphoreType.DMA,
        ],
    )
    return k(x_pad, src)


def _scatter_body(msg_hbm, dst_hbm, out_hbm, acc_sh, zbuf, idx_v, msg_v, sem):
    cid = lax.axis_index("c")
    sid = lax.axis_index("s")
    wid = sid * NC + cid

    @pl.loop(0, ZBUF_ROWS)
    def _(i):
        @pl.loop(0, D_PAD, step=16)
        def _(j):
            zbuf[i, pl.ds(j, 16)] = jnp.zeros((16,), jnp.float32)

    row0 = sid * ROWS_PER_SUB

    @pl.loop(0, ROWS_PER_SUB // ZBUF_ROWS)
    def _(r):
        pltpu.sync_copy(zbuf, acc_sh.at[pl.ds(row0 + r * ZBUF_ROWS, ZBUF_ROWS)])

    @pl.when(sid == NS - 1)
    def _():
        pltpu.sync_copy(zbuf.at[pl.ds(0, TAIL_ROWS)],
                        acc_sh.at[pl.ds(NS * ROWS_PER_SUB, TAIL_ROWS)])

    plsc.subcore_barrier()

    @pl.loop(0, MAX_CHUNKS_PER_W)
    def _(i):
        chunk = wid + i * NW

        @pl.when(chunk < N_CHUNKS)
        def _():
            base = chunk * CHUNK
            pltpu.sync_copy(dst_hbm.at[pl.ds(base, CHUNK)], idx_v)
            pltpu.sync_copy(msg_hbm.at[pl.ds(base, CHUNK)], msg_v)
            pltpu.sync_copy(msg_v, acc_sh.at[idx_v], add=True)

    plsc.subcore_barrier()
    pltpu.sync_copy(acc_sh.at[pl.ds(row0, ROWS_PER_SUB)],
                    out_hbm.at[cid, pl.ds(row0, ROWS_PER_SUB)])

    @pl.when(sid == NS - 1)
    def _():
        pltpu.sync_copy(acc_sh.at[pl.ds(NS * ROWS_PER_SUB, TAIL_ROWS)],
                        out_hbm.at[cid, pl.ds(NS * ROWS_PER_SUB, TAIL_ROWS)])


def _sc_segment_sum(msg, dst):
    k = pl.kernel(
        _scatter_body,
        out_type=jax.ShapeDtypeStruct((NC, N_NODES, D_PAD), jnp.float32),
        mesh=_sc_mesh(),
        scratch_types=[
            pltpu.VMEM_SHARED((N_NODES, D_PAD), jnp.float32),
            pltpu.VMEM((ZBUF_ROWS, D_PAD), jnp.float32),
            pltpu.VMEM((CHUNK,), jnp.int32),
            pltpu.VMEM((CHUNK, D_PAD), jnp.float32),
            pltpu.SemaphoreType.DMA,
        ],
    )
    return k(msg, dst)


E_BLK_A = 800


def _edge_mlp_body(ea_ref, we1, be1, we2, be2, ge, bbe, eps_e, wf1, t_ref):
    ea = ea_ref[...] * (1.0 + eps_e[0, 0])
    h1 = ea @ we1[...] + be1[...]
    h1 = jnp.where(h1 > 0, h1, 0.01 * h1)
    he = h1 @ we2[...] + be2[...]
    mu = jnp.mean(he, axis=-1, keepdims=True)
    var = jnp.mean((he - mu) ** 2, axis=-1, keepdims=True)
    he = (he - mu) * lax.rsqrt(var + 1e-5) * ge[...] + bbe[...]
    t_ref[...] = jnp.maximum(he @ wf1[...], 0.0)


def _tc_edge_mlp(edge_attr, W_e1, b_e1, W_e2, b_e2, g_e, bb_e, eps_e, W_f1):
    full = lambda s: pl.BlockSpec(s, lambda i: (0,) * len(s))
    return pl.pallas_call(
        _edge_mlp_body,
        grid=(N_EDGES // E_BLK_A,),
        in_specs=[
            pl.BlockSpec((E_BLK_A, D_EDGE), lambda i: (i, 0)),
            full((D_EDGE, D_HID)), full((1, D_HID)),
            full((D_HID, D_HID)), full((1, D_HID)),
            full((1, D_HID)), full((1, D_HID)), full((1, 1)),
            full((D_HID, D_NODE)),
        ],
        out_specs=pl.BlockSpec((E_BLK_A, D_NODE), lambda i: (i, 0)),
        out_shape=jax.ShapeDtypeStruct((N_EDGES, D_NODE), jnp.float32),
    )(edge_attr, W_e1, b_e1, W_e2, b_e2, g_e, bb_e, eps_e, W_f1)


E_BLK_B = 640


def _msg_body(t_ref, h_ref, erep, etile, w2r, eps_n, msg_ref):
    t = t_ref[...]
    h = h_ref[...] * (1.0 + eps_n[0, 0])
    t_rep = jnp.dot(t, erep[...], preferred_element_type=jnp.float32)
    h_tile = jnp.dot(h, etile[...], preferred_element_type=jnp.float32)
    z = t_rep * h_tile
    msg_ref[...] = jnp.dot(z, w2r[...], preferred_element_type=jnp.float32)


def _tc_msg(t, h_src, Erep, Etile_pad, W2r_pad, eps_n):
    full = lambda s: pl.BlockSpec(s, lambda i: (0,) * len(s))
    return pl.pallas_call(
        _msg_body,
        grid=(N_EDGES // E_BLK_B,),
        in_specs=[
            pl.BlockSpec((E_BLK_B, D_HID), lambda i: (i, 0)),
            pl.BlockSpec((E_BLK_B, D_PAD), lambda i: (i, 0)),
            full((D_HID, D_HID * D_NODE)),
            full((D_PAD, D_HID * D_NODE)),
            full((D_HID * D_NODE, D_PAD)),
            full((1, 1)),
        ],
        out_specs=pl.BlockSpec((E_BLK_B, D_PAD), lambda i: (i, 0)),
        out_shape=jax.ShapeDtypeStruct((N_EDGES, D_PAD), jnp.float32),
    )(t, h_src, Erep, Etile_pad, W2r_pad, eps_n)


N_BLK_C = 1000


def _final_body(x_ref, p0_ref, p1_ref, eps_n, wn1, bn1, wn2, bn2, gn, bbn,
                wp1, bp1, wp2, bp2, out_ref):
    i = pl.program_id(0)
    nblocks = pl.num_programs(0)

    p = p0_ref[...] + p1_ref[...]
    h = x_ref[...] * (1.0 + eps_n[0, 0]) + p[:, :D_NODE]
    h1 = h @ wn1[...] + bn1[...]
    h1 = jnp.where(h1 > 0, h1, 0.01 * h1)
    ho = h1 @ wn2[...] + bn2[...]
    mu = jnp.mean(ho, axis=-1, keepdims=True)
    var = jnp.mean((ho - mu) ** 2, axis=-1, keepdims=True)
    ho = (ho - mu) * lax.rsqrt(var + 1e-5) * gn[...] + bbn[...]
    part = jnp.sum(ho, axis=0, keepdims=True)

    @pl.when(i == 0)
    def _():
        out_ref[...] = jnp.zeros_like(out_ref)

    out_ref[...] += part

    @pl.when(i == nblocks - 1)
    def _():
        type_emb = out_ref[...] * (1.0 / N_NODES)
        w = jnp.tanh(type_emb @ wp1[...] + bp1[...]) @ wp2[...] + bp2[...]
        beta = jnp.exp(w - w)
        out_ref[...] = beta * type_emb


def _tc_final(x, p0, p1, eps_n, W_n1, b_n1, W_n2, b_n2, g_n, bb_n,
              W_p1, b_p1, W_p2, b_p2):
    full = lambda s: pl.BlockSpec(s, lambda i: (0,) * len(s))
    return pl.pallas_call(
        _final_body,
        grid=(N_NODES // N_BLK_C,),
        in_specs=[
            pl.BlockSpec((N_BLK_C, D_NODE), lambda i: (i, 0)),
            pl.BlockSpec((N_BLK_C, D_PAD), lambda i: (i, 0)),
            pl.BlockSpec((N_BLK_C, D_PAD), lambda i: (i, 0)),
            full((1, 1)),
            full((D_NODE, D_HID)), full((1, D_HID)),
            full((D_HID, D_HID)), full((1, D_HID)),
            full((1, D_HID)), full((1, D_HID)),
            full((D_HID, 128)), full((1, 128)),
            full((128, 1)), full((1, 1)),
        ],
        out_specs=pl.BlockSpec((1, D_HID), lambda i: (0, 0)),
        out_shape=jax.ShapeDtypeStruct((1, D_HID), jnp.float32),
    )(x, p0, p1, eps_n, W_n1, b_n1, W_n2, b_n2, g_n, bb_n,
      W_p1, b_p1, W_p2, b_p2)


def kernel(x, edge_index, edge_attr, W_e1, b_e1, W_e2, b_e2, g_e, bb_e, eps_e,
           W_f1, W_f2, eps_n, W_n1, b_n1, W_n2, b_n2, g_n, bb_n,
           W_p1, b_p1, W_p2, b_p2):
    src = edge_index[0]
    dst = edge_index[1]

    x_pad = jnp.zeros((N_NODES, D_PAD), jnp.float32).at[:, :D_NODE].set(x)

    eye = jnp.eye(D_HID, dtype=jnp.float32)
    Erep = jnp.repeat(eye, D_NODE, axis=1)
    Etile = jnp.tile(eye, (1, D_HID))
    Etile_pad = jnp.zeros((D_PAD, D_HID * D_NODE), jnp.float32).at[:D_NODE].set(Etile)
    W2r = W_f2.reshape(D_HID * D_NODE, D_NODE)
    W2r_pad = jnp.zeros((D_HID * D_NODE, D_PAD), jnp.float32).at[:, :D_NODE].set(W2r)

    r2 = lambda a: a.reshape(1, -1)
    eps_e2 = eps_e.reshape(1, 1)
    eps_n2 = eps_n.reshape(1, 1)

    h_src = _sc_gather(x_pad, src)
    t = _tc_edge_mlp(edge_attr, W_e1, r2(b_e1), W_e2, r2(b_e2),
                     r2(g_e), r2(bb_e), eps_e2, W_f1)
    msg = _tc_msg(t, h_src, Erep, Etile_pad, W2r_pad, eps_n2)
    partials = _sc_segment_sum(msg, dst)
    out = _tc_final(x, partials[0], partials[1], eps_n2,
                    W_n1, r2(b_n1), W_n2, r2(b_n2), r2(g_n), r2(bb_n),
                    W_p1, r2(b_p1), W_p2, r2(b_p2))
    return out
